# Initial kernel scaffold; baseline (speedup 1.0000x reference)
import jax
import jax.numpy as jnp
from jax.experimental import pallas as pl


def kernel(x, edge_index, W1, b1, W2, b2):
    raise NotImplementedError("write your pallas kernel here")


# speedup vs baseline: 10.1038x; 10.1038x over previous
import functools

import jax
import jax.numpy as jnp
from jax import lax
from jax.experimental import pallas as pl
from jax.experimental.pallas import tpu as pltpu
from jax.experimental.pallas import tpu_sc as plsc

N = 10000
E = 160000
D = 256
DH = 128
BM = 1000
NB = N // BM
CH = 128
NCHUNK = E // CH
NS = 16
NP = 10240
ROWS_PT = NP // NS
HB = 640

_mesh = plsc.VectorSubcoreMesh(core_axis_name="c", subcore_axis_name="s")


NBIN = HB * 16
SLAB = NBIN // NS


@functools.partial(
    pl.kernel,
    mesh=_mesh,
    out_type=jax.ShapeDtypeStruct((NBIN,), jnp.float32),
    compiler_params=pltpu.CompilerParams(needs_layout_passes=False),
    scratch_types=[
        pltpu.VMEM((NBIN,), jnp.float32),
        pltpu.VMEM((400,), jnp.int32),
        pltpu.VMEM((SLAB,), jnp.float32),
        pltpu.VMEM((SLAB,), jnp.float32),
        pltpu.VMEM_SHARED((NS, NBIN), jnp.float32),
    ],
)
def _deg_kernel(dst_hbm, out_hbm, hist, dstv, accv, tmpv, sh):
    c = lax.axis_index("c")
    s = lax.axis_index("s")

    @pl.when(c == 0)
    def _core0():
        def _z(i, carry):
            hist[pl.ds(i * 16, 16)] = jnp.zeros((16,), jnp.float32)
            return carry

        lax.fori_loop(0, NBIN // 16, _z, None)

        def _outer(k, carry):
            off = s * (E // NS) + k * 400
            pltpu.sync_copy(dst_hbm.at[pl.ds(off, 400)], dstv)

            def _inner(j, c2):
                d16 = dstv[pl.ds(j * 16, 16)]
                plsc.addupdate_scatter(hist, [d16],
                                       jnp.ones((16,), jnp.float32))
                return c2

            lax.fori_loop(0, 25, _inner, None)
            return carry

        lax.fori_loop(0, 25, _outer, None)
        pltpu.sync_copy(hist, sh.at[s])
        plsc.subcore_barrier()

        def _zacc(i, carry):
            accv[pl.ds(i * 16, 16)] = jnp.zeros((16,), jnp.float32)
            return carry

        lax.fori_loop(0, SLAB // 16, _zacc, None)

        def _merge(t, carry):
            pltpu.sync_copy(sh.at[t, pl.ds(s * SLAB, SLAB)], tmpv)

            def _addm(m, c2):
                sl = pl.ds(m * 16, 16)
                accv[sl] = accv[sl] + tmpv[sl]
                return c2

            lax.fori_loop(0, SLAB // 16, _addm, None)
            return carry

        lax.fori_loop(0, NS, _merge, None)
        pltpu.sync_copy(accv, out_hbm.at[pl.ds(s * SLAB, SLAB)])


def _agg_half(s, src_hbm, dst_hbm, h_ref, out_ref, src_v, dst_v, rows_v,
              acc_sh, sem):
    def _z(t, carry):
        rows_v[t >> 3, pl.ds((t & 7) * 16, 16)] = jnp.zeros((16,),
                                                            jnp.float32)
        return carry

    lax.fori_loop(0, CH * (DH // 16), _z, None)
    base = s * ROWS_PT
    for kk in range(ROWS_PT // CH):
        pltpu.sync_copy(rows_v, acc_sh.at[pl.ds(base + kk * CH, CH)])
    plsc.subcore_barrier()

    def _body(k, carry):
        cid = s + NS * k

        @pl.when(cid < NCHUNK)
        def _go():
            off = cid * CH
            pltpu.sync_copy(src_hbm.at[pl.ds(off, CH)], src_v)
            pltpu.sync_copy(dst_hbm.at[pl.ds(off, CH)], dst_v)
            pltpu.async_copy(h_ref.at[src_v], rows_v, sem).wait()
            pltpu.sync_copy(rows_v, acc_sh.at[dst_v], add=True)

        return carry

    lax.fori_loop(0, (NCHUNK + NS - 1) // NS, _body, None)
    plsc.subcore_barrier()
    pltpu.sync_copy(acc_sh.at[pl.ds(base, ROWS_PT)],
                    out_ref.at[pl.ds(base, ROWS_PT)])


@functools.partial(
    pl.kernel,
    mesh=_mesh,
    out_type=(jax.ShapeDtypeStruct((NP, DH), jnp.float32),
              jax.ShapeDtypeStruct((NP, DH), jnp.float32)),
    compiler_params=pltpu.CompilerParams(needs_layout_passes=False),
    scratch_types=[
        pltpu.VMEM((CH,), jnp.int32),
        pltpu.VMEM((CH,), jnp.int32),
        pltpu.VMEM((CH, DH), jnp.float32),
        pltpu.VMEM_SHARED((NP, DH), jnp.float32),
        pltpu.SemaphoreType.DMA,
    ],
)
def _agg_kernel(h0_hbm, h1_hbm, src_hbm, dst_hbm, out0_hbm, out1_hbm,
                src_v, dst_v, rows_v, acc_sh, sem):
    c = lax.axis_index("c")
    s = lax.axis_index("s")

    @pl.when(c == 0)
    def _c0():
        _agg_half(s, src_hbm, dst_hbm, h0_hbm, out0_hbm, src_v, dst_v,
                  rows_v, acc_sh, sem)

    @pl.when(c == 1)
    def _c1():
        _agg_half(s, src_hbm, dst_hbm, h1_hbm, out1_hbm, src_v, dst_v,
                  rows_v, acc_sh, sem)


def _leaky(v):
    return jnp.maximum(v, 0.03 * v)


def _l1_body(x_ref, w_ref, cnt_ref, o0_ref, o1_ref):
    dinv = lax.rsqrt(cnt_ref[...] + 1.0)
    h = jnp.dot(x_ref[...], w_ref[...],
                preferred_element_type=jnp.float32) * dinv
    o0_ref[...] = h[:, :DH]
    o1_ref[...] = h[:, DH:]


_l1_call = pl.pallas_call(
    _l1_body,
    grid=(NB,),
    in_specs=[
        pl.BlockSpec((BM, D), lambda i: (i, 0)),
        pl.BlockSpec((D, D), lambda i: (0, 0)),
        pl.BlockSpec((BM, 1), lambda i: (i, 0)),
    ],
    out_specs=[
        pl.BlockSpec((BM, DH), lambda i: (i, 0)),
        pl.BlockSpec((BM, DH), lambda i: (i, 0)),
    ],
    out_shape=[jax.ShapeDtypeStruct((N, DH), jnp.float32)] * 2,
)


def _mid_body(a0_ref, a1_ref, h0_ref, h1_ref, cnt_ref, b_ref, w_ref,
              o0_ref, o1_ref):
    dinv = lax.rsqrt(cnt_ref[...] + 1.0)
    b = b_ref[...]
    y0 = _leaky(dinv * (a0_ref[...] + h0_ref[...]) + b[:, :DH])
    y1 = _leaky(dinv * (a1_ref[...] + h1_ref[...]) + b[:, DH:])
    y = jnp.concatenate([y0, y1], axis=1)
    h = jnp.dot(y, w_ref[...], preferred_element_type=jnp.float32) * dinv
    o0_ref[...] = h[:, :DH]
    o1_ref[...] = h[:, DH:]


_mid_call = pl.pallas_call(
    _mid_body,
    grid=(NB,),
    in_specs=[
        pl.BlockSpec((BM, DH), lambda i: (i, 0)),
        pl.BlockSpec((BM, DH), lambda i: (i, 0)),
        pl.BlockSpec((BM, DH), lambda i: (i, 0)),
        pl.BlockSpec((BM, DH), lambda i: (i, 0)),
        pl.BlockSpec((BM, 1), lambda i: (i, 0)),
        pl.BlockSpec((1, D), lambda i: (0, 0)),
        pl.BlockSpec((D, D), lambda i: (0, 0)),
    ],
    out_specs=[
        pl.BlockSpec((BM, DH), lambda i: (i, 0)),
        pl.BlockSpec((BM, DH), lambda i: (i, 0)),
    ],
    out_shape=[jax.ShapeDtypeStruct((N, DH), jnp.float32)] * 2,
)


def _fin_body(a0_ref, a1_ref, h0_ref, h1_ref, cnt_ref, b_ref, o_ref):
    dinv = lax.rsqrt(cnt_ref[...] + 1.0)
    b = b_ref[...]
    y0 = _leaky(dinv * (a0_ref[...] + h0_ref[...]) + b[:, :DH])
    y1 = _leaky(dinv * (a1_ref[...] + h1_ref[...]) + b[:, DH:])
    o_ref[...] = jnp.concatenate([y0, y1], axis=1)


_fin_call = pl.pallas_call(
    _fin_body,
    grid=(NB,),
    in_specs=[
        pl.BlockSpec((BM, DH), lambda i: (i, 0)),
        pl.BlockSpec((BM, DH), lambda i: (i, 0)),
        pl.BlockSpec((BM, DH), lambda i: (i, 0)),
        pl.BlockSpec((BM, DH), lambda i: (i, 0)),
        pl.BlockSpec((BM, 1), lambda i: (i, 0)),
        pl.BlockSpec((1, D), lambda i: (0, 0)),
    ],
    out_specs=pl.BlockSpec((BM, D), lambda i: (i, 0)),
    out_shape=jax.ShapeDtypeStruct((N, D), jnp.float32),
)


def kernel(x, edge_index, W1, b1, W2, b2):
    src = edge_index[0]
    dst = edge_index[1]
    cnt = _deg_kernel(dst)
    cnt_col = cnt[:N].reshape(N, 1)
    b1r = b1.reshape(1, D)
    b2r = b2.reshape(1, D)

    h0, h1 = _l1_call(x, W1, cnt_col)
    a0, a1 = _agg_kernel(h0, h1, src, dst)
    g0, g1 = _mid_call(a0, a1, h0, h1, cnt_col, b1r, W2)
    c0, c1 = _agg_kernel(g0, g1, src, dst)
    return _fin_call(c0, c1, g0, g1, cnt_col, b2r)
